# Initial kernel scaffold; baseline (speedup 1.0000x reference)
import jax
import jax.numpy as jnp
from jax.experimental import pallas as pl


def kernel(x):
    raise NotImplementedError("write your pallas kernel here")


# speedup vs baseline: 9.2862x; 9.2862x over previous
import functools

import jax
import jax.numpy as jnp
from jax.experimental import pallas as pl

_K = 64
_ROWS_PER_BLOCK = 8


def _sortable_key(x):
    b = jax.lax.bitcast_convert_type(x, jnp.uint32)
    neg = b >= jnp.uint32(0x80000000)
    return jnp.where(neg, ~b, b | jnp.uint32(0x80000000))


def _topk_mask_block(x_ref, o_ref):
    xb = x_ref[...]
    u = _sortable_key(xb)

    def body(i, p):
        bit = jnp.uint32(31) - i.astype(jnp.uint32)
        c = p | (jnp.uint32(1) << bit)
        cnt = jnp.sum((u >= c).astype(jnp.int32), axis=1, keepdims=True)
        return jnp.where(cnt >= _K, c, p)

    p0 = jnp.zeros((xb.shape[0], 1), dtype=jnp.uint32)
    p = jax.lax.fori_loop(0, 32, body, p0)
    o_ref[...] = jnp.where(u >= p, xb, jnp.zeros_like(xb))


@jax.jit
def kernel(x):
    n_rows, n_cols = x.shape
    grid = (n_rows // _ROWS_PER_BLOCK,)
    return pl.pallas_call(
        _topk_mask_block,
        grid=grid,
        in_specs=[pl.BlockSpec((_ROWS_PER_BLOCK, n_cols), lambda i: (i, 0))],
        out_specs=pl.BlockSpec((_ROWS_PER_BLOCK, n_cols), lambda i: (i, 0)),
        out_shape=jax.ShapeDtypeStruct(x.shape, x.dtype),
    )(x)


# speedup vs baseline: 12.8690x; 1.3858x over previous
import functools

import jax
import jax.numpy as jnp
from jax.experimental import pallas as pl

_K = 64
_ROWS_PER_BLOCK = 8


def _sortable_key(x):
    b = jax.lax.bitcast_convert_type(x, jnp.uint32)
    neg = b >= jnp.uint32(0x80000000)
    return jnp.where(neg, ~b, b | jnp.uint32(0x80000000))


def _topk_mask_block(x_ref, o_ref):
    xb = x_ref[...]
    u = _sortable_key(xb)

    def cond(state):
        i, _, cnt_p = state
        return jnp.logical_and(i < 32, jnp.logical_not(jnp.all(cnt_p == _K)))

    def body(state):
        i, p, cnt_p = state
        bit = jnp.uint32(31) - i.astype(jnp.uint32)
        c = p | (jnp.uint32(1) << bit)
        cnt = jnp.sum((u >= c).astype(jnp.int32), axis=1, keepdims=True)
        take = cnt >= _K
        return i + 1, jnp.where(take, c, p), jnp.where(take, cnt, cnt_p)

    p0 = jnp.zeros((xb.shape[0], 1), dtype=jnp.uint32)
    c0 = jnp.full((xb.shape[0], 1), xb.shape[1], dtype=jnp.int32)
    _, p, _ = jax.lax.while_loop(cond, body, (0, p0, c0))
    o_ref[...] = jnp.where(u >= p, xb, jnp.zeros_like(xb))


@jax.jit
def kernel(x):
    n_rows, n_cols = x.shape
    grid = (n_rows // _ROWS_PER_BLOCK,)
    return pl.pallas_call(
        _topk_mask_block,
        grid=grid,
        in_specs=[pl.BlockSpec((_ROWS_PER_BLOCK, n_cols), lambda i: (i, 0))],
        out_specs=pl.BlockSpec((_ROWS_PER_BLOCK, n_cols), lambda i: (i, 0)),
        out_shape=jax.ShapeDtypeStruct(x.shape, x.dtype),
    )(x)
